# Initial kernel scaffold; baseline (speedup 1.0000x reference)
import jax
import jax.numpy as jnp
from jax.experimental import pallas as pl


def kernel(x_nchw, w1, w2, w3, w4, vecs, fc1_w, fc1_b, fc2_w, fc2_b, fc3_w, fc3_b):
    raise NotImplementedError("write your pallas kernel here")


# speedup vs baseline: 8.1225x; 8.1225x over previous
import numpy as np

import jax
import jax.numpy as jnp
from jax.experimental import pallas as pl
from jax.experimental.pallas import tpu as pltpu

_G = 24
_LANES = 256
_DT = jnp.float32

def _ring_mask(h):
    hp = h + 2
    r = np.arange(hp * hp)
    row, col = r // hp, r % hp
    keep = (row >= 1) & (row <= h) & (col >= 1) & (col <= h)
    return keep.astype(np.float32).reshape(hp * hp, 1)


_M28 = _ring_mask(28)
_M14 = _ring_mask(14)

_OFFS28 = [(di - 1) * 30 + (dj - 1) for di in range(3) for dj in range(3)]
_OFFS14 = [(di - 1) * 16 + (dj - 1) for di in range(3) for dj in range(3)]


def _feat_kernel(x_ref, w1_ref, w2_ref, w3_ref, w4_ref, vec_ref, m28_ref,
                 m14_ref, out_ref, buf_a, buf_b0, buf_b1, buf_p, buf_c,
                 buf_d0, buf_d1):
    f32 = jnp.float32
    b1, b2 = vec_ref[0:1, :], vec_ref[1:2, :]
    bn2_s, bn2_t = vec_ref[2:3, :], vec_ref[3:4, :]
    b3, bn3_s, bn3_t = vec_ref[4:5, :], vec_ref[5:6, :], vec_ref[6:7, :]
    b4, bn4_s, bn4_t = vec_ref[7:8, :], vec_ref[8:9, :], vec_ref[9:10, :]

    r0 = 0
    while r0 < 838:
        n = min(256, 838 - r0)
        row0 = 31 + r0
        acc = jnp.zeros((n, _LANES), f32)
        for k, off in enumerate(_OFFS28):
            acc = acc + jnp.dot(x_ref[0, pl.ds(row0 + off, n), :], w1_ref[k],
                                preferred_element_type=f32)
        y = jnp.maximum(acc + b1, 0.0) * m28_ref[pl.ds(row0, n), :]
        buf_a[pl.ds(row0, n), :] = y.astype(buf_a.dtype)
        r0 += n
    z31 = jnp.zeros((31, _LANES), buf_a.dtype)
    buf_a[pl.ds(0, 31), :] = z31
    buf_a[pl.ds(869, 31), :] = z31

    r0 = 0
    while r0 < 838:
        n = min(256, 838 - r0)
        row0 = 31 + r0
        acc = jnp.zeros((n, _LANES), f32)
        for k, off in enumerate(_OFFS28):
            acc = acc + jnp.dot(buf_a[pl.ds(row0 + off, n), :], w2_ref[k],
                                preferred_element_type=f32)
        y = jnp.maximum(acc + b2, 0.0)
        buf_b0[pl.ds(row0, n), :] = y[:, 0:128].astype(buf_b0.dtype)
        buf_b1[pl.ds(row0, n), :] = y[:, 128:256].astype(buf_b1.dtype)
        r0 += n

    buf_p[...] = jnp.zeros(buf_p.shape, buf_p.dtype)
    for i in range(14):
        p0 = (2 * i + 1) * 30
        p1 = (2 * i + 2) * 30
        halves = []
        for src in (buf_b0, buf_b1):
            a = src[pl.ds(p0 + 1, 14, stride=2), :]
            b = src[pl.ds(p0 + 2, 14, stride=2), :]
            c = src[pl.ds(p1 + 1, 14, stride=2), :]
            d = src[pl.ds(p1 + 2, 14, stride=2), :]
            halves.append(jnp.maximum(jnp.maximum(a, b), jnp.maximum(c, d)))
        m = jnp.concatenate(halves, axis=1).astype(f32)
        buf_p[pl.ds((i + 1) * 16 + 1, 14), :] = (m * bn2_s + bn2_t).astype(buf_p.dtype)

    z17 = jnp.zeros((17, _LANES), buf_c.dtype)
    for h in range(2):
        acc = jnp.zeros((222, _LANES), f32)
        for k, off in enumerate(_OFFS14):
            acc = acc + jnp.dot(buf_p[pl.ds(17 + off, 222), :], w3_ref[h, k],
                                preferred_element_type=f32)
        y = (jnp.maximum(acc + b3, 0.0) * bn3_s + bn3_t) * m14_ref[pl.ds(17, 222), :]
        buf_c[pl.ds(17, 222), :] = y.astype(buf_c.dtype)
        buf_c[pl.ds(0, 17), :] = z17
        buf_c[pl.ds(239, 17), :] = z17

        for q in range(2):
            acc = jnp.zeros((222, _LANES), f32)
            for k, off in enumerate(_OFFS14):
                acc = acc + jnp.dot(buf_c[pl.ds(17 + off, 222), :], w4_ref[q, k],
                                    preferred_element_type=f32)
            y4 = jnp.maximum(acc + b4, 0.0)
            buf_d0[pl.ds(17, 222), :] = y4[:, 0:128].astype(buf_d0.dtype)
            buf_d1[pl.ds(17, 222), :] = y4[:, 128:256].astype(buf_d1.dtype)

            for i in range(7):
                p0 = (2 * i + 1) * 16
                p1 = (2 * i + 2) * 16
                halves = []
                for src in (buf_d0, buf_d1):
                    a = src[pl.ds(p0 + 1, 7, stride=2), :]
                    b = src[pl.ds(p0 + 2, 7, stride=2), :]
                    c = src[pl.ds(p1 + 1, 7, stride=2), :]
                    d = src[pl.ds(p1 + 2, 7, stride=2), :]
                    halves.append(jnp.maximum(jnp.maximum(a, b), jnp.maximum(c, d)))
                m = jnp.concatenate(halves, axis=1).astype(f32)
                yo = m * bn4_s + bn4_t
                for t in range(6):
                    out_ref[0, 12 * h + 6 * q + t, pl.ds(i * 7, 7), :] = (
                        yo[:, 40 * t:40 * t + 40])


def _cls_kernel(x_ref, w1_ref, b1_ref, w2_ref, b2_ref, w3_ref, b3_ref, o_ref):
    h = jnp.dot(x_ref[...], w1_ref[...], preferred_element_type=jnp.float32)
    h = jnp.maximum(h + b1_ref[...], 0.0)
    h = jnp.dot(h, w2_ref[...], preferred_element_type=jnp.float32)
    h = jnp.maximum(h + b2_ref[...], 0.0)
    o_ref[...] = (jnp.dot(h, w3_ref[...], preferred_element_type=jnp.float32)
                  + b3_ref[...])


def _blockdiag(w, cin, cout, g):
    eye = jnp.eye(g, dtype=w.dtype)
    bd = eye[None, :, None, :, None] * w[:, None, :, None, :]
    return bd.reshape(9, g * cin, g * cout)


def _pad_shift(bd, in_off):
    return jnp.pad(bd, ((0, 0), (in_off, _LANES - in_off - bd.shape[1]),
                        (0, _LANES - bd.shape[2])))


def _tilevec(v, c, g):
    return jnp.pad(jnp.tile(v[:c], g), (0, _LANES - c * g))


def kernel(x_nchw, w1, w2, w3, w4, vecs, fc1_w, fc1_b, fc2_w, fc2_b,
           fc3_w, fc3_b):
    n = x_nchw.shape[0]
    ngroups = -(-n // _G)
    npad = ngroups * _G

    w1bd = jnp.pad(_blockdiag(w1[:, :5].reshape(9, 1, 5), 1, 5, _G),
                   ((0, 0), (0, 0), (0, _LANES - 5 * _G))).astype(_DT)
    w2bd = _pad_shift(_blockdiag(w2[:, :5, :10], 5, 10, _G), 0).astype(_DT)
    bd3 = _blockdiag(w3[:, :10, :20], 10, 20, 12)
    w3bd = jnp.stack([_pad_shift(bd3, 0), _pad_shift(bd3, 120)]).astype(_DT)
    bd4 = _blockdiag(w4[:, :20, :40], 20, 40, 6)
    w4bd = jnp.stack([_pad_shift(bd4, 0), _pad_shift(bd4, 120)]).astype(_DT)

    vec2 = jnp.stack([
        _tilevec(vecs[0], 5, _G), _tilevec(vecs[1], 10, _G),
        _tilevec(vecs[2], 10, _G), _tilevec(vecs[3], 10, _G),
        _tilevec(vecs[4], 20, 12), _tilevec(vecs[5], 20, 12),
        _tilevec(vecs[6], 20, 12),
        _tilevec(vecs[7], 40, 6), _tilevec(vecs[8], 40, 6),
        _tilevec(vecs[9], 40, 6),
    ])

    x = x_nchw.astype(jnp.float32).reshape(n, 28, 28)
    x = jnp.pad(x, ((0, npad - n), (1, 1), (1, 1)))
    x = x.reshape(ngroups, _G, 900).transpose(0, 2, 1).astype(_DT)

    feats = pl.pallas_call(
        _feat_kernel,
        out_shape=jax.ShapeDtypeStruct((ngroups, _G, 49, 40), jnp.float32),
        grid=(ngroups,),
        in_specs=[
            pl.BlockSpec((1, 900, _G), lambda b: (b, 0, 0)),
            pl.BlockSpec((9, _G, _LANES), lambda b: (0, 0, 0)),
            pl.BlockSpec((9, _LANES, _LANES), lambda b: (0, 0, 0)),
            pl.BlockSpec((2, 9, _LANES, _LANES), lambda b: (0, 0, 0, 0)),
            pl.BlockSpec((2, 9, _LANES, _LANES), lambda b: (0, 0, 0, 0)),
            pl.BlockSpec((10, _LANES), lambda b: (0, 0)),
            pl.BlockSpec((900, 1), lambda b: (0, 0)),
            pl.BlockSpec((256, 1), lambda b: (0, 0)),
        ],
        out_specs=pl.BlockSpec((1, _G, 49, 40), lambda b: (b, 0, 0, 0)),
        scratch_shapes=[
            pltpu.VMEM((900, _LANES), _DT),
            pltpu.VMEM((900, 128), _DT),
            pltpu.VMEM((900, 128), _DT),
            pltpu.VMEM((256, _LANES), _DT),
            pltpu.VMEM((256, _LANES), _DT),
            pltpu.VMEM((256, 128), _DT),
            pltpu.VMEM((256, 128), _DT),
        ],
        compiler_params=pltpu.CompilerParams(
            dimension_semantics=("parallel",)),
    )(x, w1bd, w2bd, w3bd, w4bd, vec2,
      jnp.asarray(_M28), jnp.asarray(_M14))

    feats = feats.reshape(npad, 49 * 40)

    bm = npad
    for k in range(8, 33):
        if npad % k == 0 and (npad // k) % 8 == 0:
            bm = npad // k
            break
    steps = npad // bm
    out = pl.pallas_call(
        _cls_kernel,
        out_shape=jax.ShapeDtypeStruct((npad, 10), jnp.float32),
        grid=(steps,),
        in_specs=[
            pl.BlockSpec((bm, 1960), lambda i: (i, 0)),
            pl.BlockSpec((1960, 256), lambda i: (0, 0)),
            pl.BlockSpec((1, 256), lambda i: (0, 0)),
            pl.BlockSpec((256, 512), lambda i: (0, 0)),
            pl.BlockSpec((1, 512), lambda i: (0, 0)),
            pl.BlockSpec((512, 10), lambda i: (0, 0)),
            pl.BlockSpec((1, 10), lambda i: (0, 0)),
        ],
        out_specs=pl.BlockSpec((bm, 10), lambda i: (i, 0)),
        compiler_params=pltpu.CompilerParams(
            dimension_semantics=("parallel",)),
    )(feats, fc1_w, fc1_b, fc2_w, fc2_b, fc3_w, fc3_b)
    return out[:n]
